# Initial kernel scaffold; baseline (speedup 1.0000x reference)
import jax
import jax.numpy as jnp
from jax import lax
from jax.experimental import pallas as pl
from jax.experimental.pallas import tpu as pltpu
from jax.experimental.pallas import tpu_sc as plsc

N = 10000
E = 320000
D = 128
G = 64

NW = 32
NPW = 313
NP = NW * NPW
BUF = 2048
CHUNK = 8000
CHUNKB = 2000
CAPE = E + BUF
CAPB = N + BUF
NPG = 2

_MESH = plsc.VectorSubcoreMesh(core_axis_name="c", subcore_axis_name="s")
_NEG_INF = jnp.float32(-jnp.inf)


def _worker_id():
    return lax.axis_index("s") * 2 + lax.axis_index("c")


def _scan_compact(idx_hbm, val_hbm, out_v_hbm, out_d_hbm, out_c_hbm,
                  bs_v, bd_v, stage_i, stage_v, w, lo, hi, n_items, chunk):
    nchunks = n_items // chunk
    ngroups = chunk // 16
    lane = lax.iota(jnp.int32, 16)

    def chunk_body(c, carry):
        pltpu.sync_copy(idx_hbm.at[pl.ds(c * chunk, chunk)], stage_i)
        if val_hbm is not None:
            pltpu.sync_copy(val_hbm.at[pl.ds(c * chunk, chunk)], stage_v)

        def group_body(g, carry2):
            cnt, off = carry2
            dv = stage_i[pl.ds(g * 16, 16)]
            mask = (dv >= lo) & (dv < hi)
            pc = plsc.all_reduce_population_count(mask)[0]

            @pl.when(pc > 0)
            def _():
                if val_hbm is not None:
                    vv = stage_v[pl.ds(g * 16, 16)]
                else:
                    vv = c * chunk + g * 16 + lane
                plsc.store_compressed(bs_v.at[pl.ds(cnt, 16)], vv, mask=mask)
                plsc.store_compressed(bd_v.at[pl.ds(cnt, 16)], dv - lo,
                                      mask=mask)

            cnt2 = cnt + pc
            do_flush = cnt2 >= BUF

            @pl.when(do_flush)
            def _():
                pltpu.sync_copy(bs_v.at[pl.ds(0, BUF)],
                                out_v_hbm.at[w, pl.ds(off, BUF)])
                pltpu.sync_copy(bd_v.at[pl.ds(0, BUF)],
                                out_d_hbm.at[w, pl.ds(off, BUF)])
                bs_v[pl.ds(0, 16)] = bs_v[pl.ds(BUF, 16)]
                bd_v[pl.ds(0, 16)] = bd_v[pl.ds(BUF, 16)]

            cnt3 = jnp.where(do_flush, cnt2 - BUF, cnt2)
            off2 = jnp.where(do_flush, off + BUF, off)
            return cnt3, off2

        return lax.fori_loop(0, ngroups, group_body, carry)

    cnt, off = lax.fori_loop(0, nchunks, chunk_body,
                             (jnp.int32(0), jnp.int32(0)))
    pltpu.sync_copy(bs_v.at[pl.ds(0, BUF)], out_v_hbm.at[w, pl.ds(off, BUF)])
    pltpu.sync_copy(bd_v.at[pl.ds(0, BUF)], out_d_hbm.at[w, pl.ds(off, BUF)])
    bs_v[pl.ds(0, 16)] = jnp.zeros((16,), jnp.int32) + (off + cnt)
    pltpu.sync_copy(bs_v.at[pl.ds(0, 16)], out_c_hbm.at[w])


def _phase0_body(dst_hbm, src_hbm, batch_hbm,
                 cs_hbm, cd_hbm, cc_hbm, ps_hbm, pd_hbm, pc_hbm,
                 stage_i, stage_v, bs_v, bd_v):
    w = _worker_id()
    _scan_compact(dst_hbm, src_hbm, cs_hbm, cd_hbm, cc_hbm,
                  bs_v, bd_v, stage_i, stage_v, w,
                  w * NPW, (w + 1) * NPW, E, CHUNK)
    _scan_compact(batch_hbm, None, ps_hbm, pd_hbm, pc_hbm,
                  bs_v, bd_v, stage_i, stage_v, w,
                  w * NPG, (w + 1) * NPG, N, CHUNKB)


_phase0 = pl.kernel(
    _phase0_body,
    out_type=(
        jax.ShapeDtypeStruct((NW, CAPE), jnp.int32),
        jax.ShapeDtypeStruct((NW, CAPE), jnp.int32),
        jax.ShapeDtypeStruct((NW, 16), jnp.int32),
        jax.ShapeDtypeStruct((NW, CAPB), jnp.int32),
        jax.ShapeDtypeStruct((NW, CAPB), jnp.int32),
        jax.ShapeDtypeStruct((NW, 16), jnp.int32),
    ),
    mesh=_MESH,
    scratch_types=[
        pltpu.VMEM((CHUNK,), jnp.int32),
        pltpu.VMEM((CHUNK,), jnp.int32),
        pltpu.VMEM((BUF + 16,), jnp.int32),
        pltpu.VMEM((BUF + 16,), jnp.int32),
    ],
    name="sc_phase0",
)


def _make_segmax(npw, out_rows, name):
    def body(h_hbm, cs_hbm, cd_hbm, cc_hbm, out_hbm,
             csv, cdv, cntv, rows, acc, sems):
        w = _worker_id()
        lane = lax.iota(jnp.int32, 16)
        zero_idx = jnp.zeros((16,), jnp.int32)

        pltpu.sync_copy(cc_hbm.at[w], cntv)
        total = cntv[0]

        def init_body(i, _):
            acc[i // 8, pl.ds((i % 8) * 16, 16)] = jnp.full(
                (16,), _NEG_INF, jnp.float32)
            return 0

        lax.fori_loop(0, (npw + 1) * 8, init_body, 0)

        nchunks = (total + BUF - 1) // BUF

        def fire(j, base, slot):
            g0 = j * 16
            pos = base + g0 + lane
            valid = pos < total
            idxv = jnp.where(valid, csv[pl.ds(g0, 16)], 0)
            pltpu.async_copy(h_hbm.at[idxv], rows.at[slot], sems.at[slot])

        def wait(slot):
            pltpu.make_async_copy(h_hbm.at[zero_idx], rows.at[slot],
                                  sems.at[slot]).wait()

        def chunk_body(c, _):
            base = c * BUF
            pltpu.sync_copy(cs_hbm.at[w, pl.ds(base, BUF)], csv)
            pltpu.sync_copy(cd_hbm.at[w, pl.ds(base, BUF)], cdv)
            ng = (jnp.minimum(total - base, BUF) + 15) // 16

            @pl.when(ng > 0)
            def _():
                fire(0, base, 0)

                def group_body(j, _):
                    slot = lax.rem(j, 2)

                    @pl.when(j + 1 < ng)
                    def _():
                        fire(j + 1, base, 1 - slot)

                    wait(slot)
                    g0 = j * 16
                    pos = base + g0 + lane
                    valid = pos < total
                    dloc = jnp.where(valid, cdv[pl.ds(g0, 16)],
                                     jnp.int32(npw))
                    cdv[pl.ds(g0, 16)] = dloc

                    def lane_body(l, _):
                        dl = cdv[g0 + l]
                        for k in range(8):
                            sl = pl.ds(k * 16, 16)
                            acc[dl, sl] = jnp.maximum(acc[dl, sl],
                                                      rows[slot, l, sl])
                        return 0

                    lax.fori_loop(0, 16, lane_body, 0, unroll=True)
                    return 0

                lax.fori_loop(0, ng, group_body, 0)

            return 0

        lax.fori_loop(0, nchunks, chunk_body, 0)

        def fix_body(i, _):
            sl = pl.ds((i % 8) * 16, 16)
            v = acc[i // 8, sl]
            acc[i // 8, sl] = jnp.where(v == _NEG_INF, jnp.float32(0.0), v)
            return 0

        lax.fori_loop(0, npw * 8, fix_body, 0)
        pltpu.sync_copy(acc.at[pl.ds(0, npw)], out_hbm.at[pl.ds(w * npw, npw)])

    return pl.kernel(
        body,
        out_type=jax.ShapeDtypeStruct((out_rows, D), jnp.float32),
        mesh=_MESH,
        scratch_types=[
            pltpu.VMEM((BUF,), jnp.int32),
            pltpu.VMEM((BUF,), jnp.int32),
            pltpu.VMEM((16,), jnp.int32),
            pltpu.VMEM((2, 16, D), jnp.float32),
            pltpu.VMEM((npw + 1, D), jnp.float32),
            pltpu.SemaphoreType.DMA((2,)),
        ],
        name=name,
    )


_segmax_layer = _make_segmax(NPW, NP, "sc_segmax_layer")
_segmax_pool = _make_segmax(NPG, G, "sc_segmax_pool")


def _mm_relu_body(h_ref, w_ref, b_ref, o_ref):
    o_ref[...] = jnp.maximum(
        jnp.dot(h_ref[...], w_ref[...],
                preferred_element_type=jnp.float32) + b_ref[...],
        0.0)


def _mm_relu(h, W, b):
    rows = h.shape[0]
    br = 1252
    return pl.pallas_call(
        _mm_relu_body,
        grid=(rows // br,),
        in_specs=[
            pl.BlockSpec((br, D), lambda i: (i, 0)),
            pl.BlockSpec((D, D), lambda i: (0, 0)),
            pl.BlockSpec((1, D), lambda i: (0, 0)),
        ],
        out_specs=pl.BlockSpec((br, D), lambda i: (i, 0)),
        out_shape=jax.ShapeDtypeStruct((rows, D), jnp.float32),
    )(h, W, b.reshape(1, D))


def _final_body(g_ref, wl_ref, bl_ref, wo_ref, bo_ref, o_ref):
    t = jnp.dot(g_ref[...], wl_ref[...],
                preferred_element_type=jnp.float32) + bl_ref[...]
    o_ref[...] = jnp.dot(t, wo_ref[...],
                         preferred_element_type=jnp.float32) + bo_ref[...]


def _final(g, Wl, bl, Wo, bo):
    return pl.pallas_call(
        _final_body,
        out_shape=jax.ShapeDtypeStruct((G, Wo.shape[1]), jnp.float32),
    )(g, Wl, bl.reshape(1, -1), Wo, bo.reshape(1, -1))


def kernel(x, edge_index, batch, W1, b1, W2, b2, W3, b3, Wl, bl, Wo, bo):
    src = edge_index[0]
    dst = edge_index[1]
    cs, cd, cc, ps, pd, pc = _phase0(dst, src, batch)
    x_p = jnp.zeros((NP, D), jnp.float32).at[:N].set(x)
    h = x_p
    for W, b in ((W1, b1), (W2, b2), (W3, b3)):
        agg = _segmax_layer(h, cs, cd, cc)
        h = _mm_relu(agg, W, b)
    g = _segmax_pool(h, ps, pd, pc)
    return _final(g, Wl, bl, Wo, bo)


# speedup vs baseline: 2.5447x; 2.5447x over previous
import jax
import jax.numpy as jnp
from jax import lax
from jax.experimental import pallas as pl
from jax.experimental.pallas import tpu as pltpu
from jax.experimental.pallas import tpu_sc as plsc

N = 10000
E = 320000
D = 128
G = 64

NW = 32
NPW = 320
NP = NW * NPW
BUF = 2048
CHUNK = 8000
CHUNKB = 2000
CAPE = E + BUF
CAPB = N + BUF
NPG = 2

_MESH = plsc.VectorSubcoreMesh(core_axis_name="c", subcore_axis_name="s")
_NEG_INF = float("-inf")


def _al8(i):
    return pl.multiple_of(i, 8)


def _worker_id():
    return lax.axis_index("s") * 2 + lax.axis_index("c")


def _scan_compact(idx_hbm, val_hbm, out_v_hbm, out_d_hbm, out_c_hbm,
                  bs_v, bd_v, stage_i, stage_v, w, wbase, lo, hi, n_items,
                  chunk):
    nchunks = n_items // chunk
    ngroups = chunk // 16
    lane = lax.iota(jnp.int32, 16)

    def chunk_body(c, carry):
        pltpu.sync_copy(idx_hbm.at[pl.ds(c * chunk, chunk)],
                        stage_i.at[pl.ds(0, chunk)])
        if val_hbm is not None:
            pltpu.sync_copy(val_hbm.at[pl.ds(c * chunk, chunk)],
                            stage_v.at[pl.ds(0, chunk)])

        def group_body(g, carry2):
            cnt, off = carry2
            dv = stage_i[pl.ds(g * 16, 16)]
            mask = (dv >= lo) & (dv < hi)
            pc = plsc.all_reduce_population_count(mask)[0]

            @pl.when(pc > 0)
            def _():
                if val_hbm is not None:
                    vv = stage_v[pl.ds(g * 16, 16)]
                else:
                    vv = c * chunk + g * 16 + lane
                plsc.store_compressed(bs_v.at[pl.ds(cnt, 16)], vv, mask=mask)
                plsc.store_compressed(bd_v.at[pl.ds(cnt, 16)], dv - lo,
                                      mask=mask)

            cnt2 = cnt + pc
            do_flush = cnt2 >= BUF

            @pl.when(do_flush)
            def _():
                pltpu.sync_copy(bs_v.at[pl.ds(0, BUF)],
                                out_v_hbm.at[pl.ds(_al8(wbase + off), BUF)])
                pltpu.sync_copy(bd_v.at[pl.ds(0, BUF)],
                                out_d_hbm.at[pl.ds(_al8(wbase + off), BUF)])
                bs_v[pl.ds(0, 16)] = bs_v[pl.ds(BUF, 16)]
                bd_v[pl.ds(0, 16)] = bd_v[pl.ds(BUF, 16)]

            cnt3 = jnp.where(do_flush, cnt2 - BUF, cnt2)
            off2 = jnp.where(do_flush, off + BUF, off)
            return cnt3, off2

        return lax.fori_loop(0, ngroups, group_body, carry)

    cnt, off = lax.fori_loop(0, nchunks, chunk_body,
                             (jnp.int32(0), jnp.int32(0)))
    pltpu.sync_copy(bs_v.at[pl.ds(0, BUF)],
                    out_v_hbm.at[pl.ds(_al8(wbase + off), BUF)])
    pltpu.sync_copy(bd_v.at[pl.ds(0, BUF)],
                    out_d_hbm.at[pl.ds(_al8(wbase + off), BUF)])
    bs_v[pl.ds(0, 16)] = jnp.zeros((16,), jnp.int32) + (off + cnt)
    pltpu.sync_copy(bs_v.at[pl.ds(0, 16)], out_c_hbm.at[pl.ds(_al8(w * 16), 16)])


def _phase0_body(dst_hbm, src_hbm, batch_hbm,
                 cs_hbm, cd_hbm, cc_hbm, ps_hbm, pd_hbm, pc_hbm,
                 stage_i, stage_v, bs_v, bd_v):
    w = _worker_id()
    _scan_compact(dst_hbm, src_hbm, cs_hbm, cd_hbm, cc_hbm,
                  bs_v, bd_v, stage_i, stage_v, w, w * CAPE,
                  w * NPW, (w + 1) * NPW, E, CHUNK)
    _scan_compact(batch_hbm, None, ps_hbm, pd_hbm, pc_hbm,
                  bs_v, bd_v, stage_i, stage_v, w, w * CAPB,
                  w * NPG, (w + 1) * NPG, N, CHUNKB)


_phase0 = pl.kernel(
    _phase0_body,
    out_type=(
        jax.ShapeDtypeStruct((NW * CAPE,), jnp.int32),
        jax.ShapeDtypeStruct((NW * CAPE,), jnp.int32),
        jax.ShapeDtypeStruct((NW * 16,), jnp.int32),
        jax.ShapeDtypeStruct((NW * CAPB,), jnp.int32),
        jax.ShapeDtypeStruct((NW * CAPB,), jnp.int32),
        jax.ShapeDtypeStruct((NW * 16,), jnp.int32),
    ),
    mesh=_MESH,
    compiler_params=pltpu.CompilerParams(needs_layout_passes=False),
    scratch_types=[
        pltpu.VMEM((CHUNK,), jnp.int32),
        pltpu.VMEM((CHUNK,), jnp.int32),
        pltpu.VMEM((BUF + 16,), jnp.int32),
        pltpu.VMEM((BUF + 16,), jnp.int32),
    ],
    name="sc_phase0",
)


def _make_segmax(npw, out_rows, cap, name, out_stride=None):
    if out_stride is None:
        out_stride = npw
    nacc = max(npw, out_stride) + 1

    def body(h_hbm, cs_hbm, cd_hbm, cc_hbm, out_hbm,
             csv, cdv, cntv, rows, acc, sems):
        w = _worker_id()
        lane = lax.iota(jnp.int32, 16)
        zero_idx = jnp.zeros((16,), jnp.int32)

        pltpu.sync_copy(cc_hbm.at[pl.ds(_al8(w * 16), 16)], cntv)
        total = cntv[...][0]

        def init_body(i, _):
            acc[i // 8, pl.ds((i % 8) * 16, 16)] = jnp.full(
                (16,), _NEG_INF, jnp.float32)
            return 0

        lax.fori_loop(0, nacc * 8, init_body, 0)

        nchunks = (total + BUF - 1) // BUF

        def fire(j, base, slot):
            g0 = j * 16
            pos = base + g0 + lane
            valid = pos < total
            idxv = jnp.where(valid, csv[pl.ds(g0, 16)], 0)
            pltpu.async_copy(h_hbm.at[idxv], rows.at[slot], sems.at[slot])

        def wait(slot):
            pltpu.make_async_copy(h_hbm.at[zero_idx], rows.at[slot],
                                  sems.at[slot]).wait()

        def chunk_body(c, _):
            base = c * BUF
            pltpu.sync_copy(cs_hbm.at[pl.ds(_al8(w * cap + base), BUF)], csv)
            pltpu.sync_copy(cd_hbm.at[pl.ds(_al8(w * cap + base), BUF)], cdv)
            ng = (jnp.minimum(total - base, BUF) + 15) // 16

            @pl.when(ng > 0)
            def _():
                fire(0, base, 0)

                def group_body(j, _):
                    slot = lax.rem(j, 2)

                    @pl.when(j + 1 < ng)
                    def _():
                        fire(j + 1, base, 1 - slot)

                    wait(slot)
                    g0 = j * 16
                    pos = base + g0 + lane
                    valid = pos < total
                    dloc = jnp.where(valid, cdv[pl.ds(g0, 16)],
                                     jnp.int32(npw))

                    for l in range(16):
                        dl = dloc[l]
                        for k in range(8):
                            sl = pl.ds(k * 16, 16)
                            acc[dl, sl] = jnp.maximum(acc[dl, sl],
                                                      rows[slot, l, sl])
                    return 0

                lax.fori_loop(0, ng, group_body, 0)

            return 0

        lax.fori_loop(0, nchunks, chunk_body, 0)

        def fix_body(i, _):
            sl = pl.ds((i % 8) * 16, 16)
            v = acc[i // 8, sl]
            acc[i // 8, sl] = jnp.where(v == _NEG_INF, jnp.float32(0.0), v)
            return 0

        lax.fori_loop(0, out_stride * 8, fix_body, 0)
        pltpu.sync_copy(acc.at[pl.ds(0, out_stride)],
                        out_hbm.at[pl.ds(_al8(w * out_stride), out_stride)])

    return pl.kernel(
        body,
        out_type=jax.ShapeDtypeStruct((out_rows, D), jnp.float32),
        mesh=_MESH,
        compiler_params=pltpu.CompilerParams(needs_layout_passes=False),
        scratch_types=[
            pltpu.VMEM((BUF,), jnp.int32),
            pltpu.VMEM((BUF,), jnp.int32),
            pltpu.VMEM((16,), jnp.int32),
            pltpu.VMEM((2, 16, D), jnp.float32),
            pltpu.VMEM((nacc, D), jnp.float32),
            pltpu.SemaphoreType.DMA((2,)),
        ],
        name=name,
    )


_segmax_layer = _make_segmax(NPW, NP, CAPE, "sc_segmax_layer")
_segmax_pool = _make_segmax(NPG, NW * 8, CAPB, "sc_segmax_pool",
                            out_stride=8)


def _mm_relu_body(h_ref, w_ref, b_ref, o_ref):
    o_ref[...] = jnp.maximum(
        jnp.dot(h_ref[...], w_ref[...],
                preferred_element_type=jnp.float32) + b_ref[...],
        0.0)


def _mm_relu(h, W, b):
    rows = h.shape[0]
    br = 2504
    return pl.pallas_call(
        _mm_relu_body,
        grid=(rows // br,),
        in_specs=[
            pl.BlockSpec((br, D), lambda i: (i, 0)),
            pl.BlockSpec((D, D), lambda i: (0, 0)),
            pl.BlockSpec((1, D), lambda i: (0, 0)),
        ],
        out_specs=pl.BlockSpec((br, D), lambda i: (i, 0)),
        out_shape=jax.ShapeDtypeStruct((rows, D), jnp.float32),
    )(h, W, b.reshape(1, D))


def _final_body(g_ref, wl_ref, bl_ref, wo_ref, bo_ref, o_ref):
    t = jnp.dot(g_ref[...], wl_ref[...],
                preferred_element_type=jnp.float32) + bl_ref[...]
    o_ref[...] = jnp.dot(t, wo_ref[...],
                         preferred_element_type=jnp.float32) + bo_ref[...]


def _final(g, Wl, bl, Wo, bo):
    return pl.pallas_call(
        _final_body,
        out_shape=jax.ShapeDtypeStruct((G, Wo.shape[1]), jnp.float32),
    )(g, Wl, bl.reshape(1, -1), Wo, bo.reshape(1, -1))


def kernel(x, edge_index, batch, W1, b1, W2, b2, W3, b3, Wl, bl, Wo, bo):
    src = edge_index[0]
    dst = edge_index[1]
    cs, cd, cc, ps, pd, pc = _phase0(dst, src, batch)
    x_p = jnp.zeros((NP, D), jnp.float32).at[:N].set(x)
    h = x_p
    for W, b in ((W1, b1), (W2, b2), (W3, b3)):
        agg = _segmax_layer(h, cs, cd, cc)
        h = _mm_relu(agg, W, b)
    g8 = _segmax_pool(h, ps, pd, pc)
    g = g8.reshape(NW, 8, D)[:, :NPG].reshape(G, D)
    return _final(g, Wl, bl, Wo, bo)
